# Initial kernel scaffold; baseline (speedup 1.0000x reference)
import jax
import jax.numpy as jnp
from jax.experimental import pallas as pl


def kernel(x, edge_index, batch_idx, atom_emb, conv_Wl, conv_bl, conv_Wr, bn_gamma, bn_beta, pool_Wl, pool_bl, pool_Wr, assign_Wl, assign_bl, assign_Wr, lin_W, lin_b):
    raise NotImplementedError("write your pallas kernel here")


# speedup vs baseline: 17.2090x; 17.2090x over previous
import functools

import jax
import jax.numpy as jnp
import numpy as np
from jax import lax
from jax.experimental import pallas as pl
from jax.experimental.pallas import tpu as pltpu
from jax.experimental.pallas import tpu_sc as plsc

_N = 10000
_E = 320000
_G = 400
_PN = 25
_C = 128
_CL = 7

_OFFSETS = np.concatenate([[0], np.cumsum([119, 5, 12, 12, 10, 6, 6, 2, 2])[:-1]]).astype(np.int32)
_TBL = 174

_NC, _NS = 2, 16
_EPT = 10240
_ROWS = _EPT // 128
_EPAD = _EPT * _NC * _NS
_SH = 250112
_SLC = _SH // _NS


def _adj_body(src_hbm, dst_hbm, out_hbm, src_v, dst_v, idx_v, ones_v, z_v, acc_sh):
    c = lax.axis_index("c")
    s = lax.axis_index("s")
    wid = c * _NS + s
    base = wid * _EPT

    zero16 = jnp.zeros((16,), jnp.float32)

    def zbody(j, carry):
        z_v[pl.ds(j * 16, 16)] = zero16
        return carry

    lax.fori_loop(0, _SLC // 16, zbody, 0)
    pltpu.sync_copy(z_v, acc_sh.at[pl.ds(s * _SLC, _SLC)])

    pltpu.sync_copy(src_hbm.at[pl.ds(base, _EPT)], src_v)
    pltpu.sync_copy(dst_hbm.at[pl.ds(base, _EPT)], dst_v)

    ones16 = jnp.ones((16,), jnp.float32)
    for j in range(8):
        ones_v[pl.ds(j * 16, 16)] = ones16

    def cbody(v, carry):
        sv = src_v[pl.ds(v * 16, 16)]
        dv = dst_v[pl.ds(v * 16, 16)]
        iv = sv * 25 + lax.rem(dv, 25)
        idx_v[v // 8, pl.ds(lax.rem(v, 8) * 16, 16)] = iv
        return carry

    lax.fori_loop(0, _ROWS * 8, cbody, 0)

    plsc.subcore_barrier()

    def sbody(r, carry):
        pltpu.sync_copy(ones_v, acc_sh.at[idx_v.at[r]], add=True)
        return carry

    lax.fori_loop(0, _ROWS, sbody, 0)

    plsc.subcore_barrier()
    pltpu.sync_copy(acc_sh.at[pl.ds(s * _SLC, _SLC)], z_v)
    pltpu.sync_copy(z_v, out_hbm.at[pl.ds(c * _SH + s * _SLC, _SLC)])


@functools.cache
def _adj_sc_kernel():
    return pl.kernel(
        _adj_body,
        out_type=jax.ShapeDtypeStruct((_NC * _SH,), jnp.float32),
        scratch_types=[
            pltpu.VMEM((_EPT,), jnp.int32),
            pltpu.VMEM((_EPT,), jnp.int32),
            pltpu.VMEM((_ROWS, 128), jnp.int32),
            pltpu.VMEM((128,), jnp.float32),
            pltpu.VMEM((_SLC,), jnp.float32),
            pltpu.VMEM_SHARED((_SH,), jnp.float32),
        ],
        mesh=plsc.VectorSubcoreMesh(core_axis_name="c", subcore_axis_name="s",
                                    num_cores=_NC, num_subcores=_NS),
    )


_GB = 16
_NB = _GB * _PN
_GQ = _GB * _CL
_NBLK = _G // _GB
_INV_SQRT = float(1.0 / np.sqrt(1.0 + 1e-05))


def _dot(a, b):
    return lax.dot_general(a, b, (((1,), (0,)), ((), ())),
                           precision=lax.Precision.HIGHEST,
                           preferred_element_type=jnp.float32)


def _dot_nt(a, b):
    return lax.dot_general(a, b, (((1,), (1,)), ((), ())),
                           precision=lax.Precision.HIGHEST,
                           preferred_element_type=jnp.float32)


def _dot_bf(a, b):
    return lax.dot_general(a.astype(jnp.bfloat16), b.astype(jnp.bfloat16),
                           (((1,), (0,)), ((), ())),
                           preferred_element_type=jnp.float32)


def _tc_body(x_ref, adj_ref, emb_ref, Wl_ref, bl_ref, Wr_ref, gam_ref, bet_ref,
             aWl_ref, abl_ref, aWr_ref, lW_ref, lb_ref,
             probs_ref, link_ref, ent_ref):
    i = pl.program_id(0)

    xi = x_ref[...]
    t_iota = lax.broadcasted_iota(jnp.int32, (_NB, _TBL), 1)
    oh = jnp.zeros((_NB, _TBL), jnp.float32)
    for f in range(9):
        oh = oh + (t_iota == xi[:, f][:, None]).astype(jnp.float32)
    h = _dot(oh, emb_ref[...])

    adjA = adj_ref[0] + adj_ref[1]
    r2 = lax.broadcasted_iota(jnp.int32, (_NB, _NB), 0)
    c2 = lax.broadcasted_iota(jnp.int32, (_NB, _NB), 1)
    samegraph = (r2 // _PN == c2 // _PN).astype(jnp.float32)
    rr = lax.broadcasted_iota(jnp.int32, (_NB, _PN), 0)
    kk = lax.broadcasted_iota(jnp.int32, (_NB, _PN), 1)
    oh_dmod = (lax.rem(rr, _PN) == kk).astype(jnp.float32)
    bdT = _dot_nt(oh_dmod, adjA) * samegraph
    denom = jnp.maximum(jnp.sum(bdT, axis=1, keepdims=True), 1.0)

    for li in range(2):
        res = h
        mean = _dot(bdT, h) / denom
        h = _dot(mean, Wl_ref[li]) + bl_ref[li][None, :] + _dot(h, Wr_ref[li])
        h = gam_ref[li][None, :] * h * _INV_SQRT + bet_ref[li][None, :]
        h = jnp.maximum(h, 0.0) + res

    meanA = _dot(bdT, h) / denom
    sr = _dot(meanA, aWl_ref[...]) + abl_ref[...] + _dot(h, aWr_ref[...])
    sr = sr - jnp.max(sr, axis=1, keepdims=True)
    se = jnp.exp(sr)
    S = se / jnp.sum(se, axis=1, keepdims=True)

    bdA = _dot_nt(adjA, oh_dmod)
    Q = _dot_nt(S, S)
    dlt = (bdA - Q) * samegraph
    linkpart = jnp.sum(dlt * dlt, keepdims=True).reshape(1, 1)
    entpart = jnp.sum(-S * jnp.log(S + 1e-15), keepdims=True).reshape(1, 1)

    qq = lax.broadcasted_iota(jnp.int32, (_GQ, _CL), 0)
    kk7 = lax.broadcasted_iota(jnp.int32, (_GQ, _CL), 1)
    oh7 = (lax.rem(qq, _CL) == kk7).astype(jnp.float32)
    qr = lax.broadcasted_iota(jnp.int32, (_GQ, _NB), 0)
    rc = lax.broadcasted_iota(jnp.int32, (_GQ, _NB), 1)
    qmask = (qr // _CL == rc // _PN).astype(jnp.float32)
    P = _dot_nt(oh7, S) * qmask
    h2 = _dot(P, h)

    q2r = lax.broadcasted_iota(jnp.int32, (_GQ, _GQ), 0)
    q2c = lax.broadcasted_iota(jnp.int32, (_GQ, _GQ), 1)
    Mm = (q2r // _CL == q2c // _CL).astype(jnp.float32) * (1.0 / _CL)
    for li in (2, 3):
        res = h2
        mean2 = _dot(Mm, h2)
        h2 = _dot_bf(mean2, Wl_ref[li]) + bl_ref[li][None, :] + _dot_bf(h2, Wr_ref[li])
        h2 = gam_ref[li][None, :] * h2 * _INV_SQRT + bet_ref[li][None, :]
        h2 = h2 + res

    gr = lax.broadcasted_iota(jnp.int32, (_GB, _GQ), 0)
    gc = lax.broadcasted_iota(jnp.int32, (_GB, _GQ), 1)
    R = (gc // _CL == gr).astype(jnp.float32) * (1.0 / _CL)
    hg = _dot(R, h2)
    logit = _dot(hg, lW_ref[...]) + lb_ref[...]
    probs_ref[...] = 1.0 / (1.0 + jnp.exp(-logit))

    @pl.when(i == 0)
    def _():
        link_ref[...] = jnp.zeros((1, 1), jnp.float32)
        ent_ref[...] = jnp.zeros((1, 1), jnp.float32)

    link_ref[...] += linkpart
    ent_ref[...] += entpart

    @pl.when(i == _NBLK - 1)
    def _():
        link_ref[...] = jnp.sqrt(link_ref[...]) * (1.0 / float(_G * _PN * _PN))
        ent_ref[...] = ent_ref[...] * (1.0 / float(_N))


def _dense_tc(x, adjp, atom_emb, conv_Wl, conv_bl, conv_Wr, bn_gamma, bn_beta,
              assign_Wl, assign_bl, assign_Wr, lin_W, lin_b):
    full = lambda shp: pl.BlockSpec(shp, lambda i: tuple(0 for _ in shp))
    probs, link, ent = pl.pallas_call(
        _tc_body,
        grid=(_NBLK,),
        in_specs=[
            pl.BlockSpec((_NB, 9), lambda i: (i, 0)),
            pl.BlockSpec((_NC, _NB, _PN), lambda i: (0, i, 0)),
            full((_TBL, _C)),
            full((4, _C, _C)),
            full((4, _C)),
            full((4, _C, _C)),
            full((4, _C)),
            full((4, _C)),
            full((_C, _CL)),
            full((1, _CL)),
            full((_C, _CL)),
            full((_C, 1)),
            full((1, 1)),
        ],
        out_specs=[
            pl.BlockSpec((_GB, 1), lambda i: (i, 0)),
            pl.BlockSpec((1, 1), lambda i: (0, 0)),
            pl.BlockSpec((1, 1), lambda i: (0, 0)),
        ],
        out_shape=[
            jax.ShapeDtypeStruct((_G, 1), jnp.float32),
            jax.ShapeDtypeStruct((1, 1), jnp.float32),
            jax.ShapeDtypeStruct((1, 1), jnp.float32),
        ],
    )(x, adjp, atom_emb, conv_Wl, conv_bl, conv_Wr, bn_gamma, bn_beta,
      assign_Wl, assign_bl.reshape(1, _CL), assign_Wr, lin_W, lin_b.reshape(1, 1))
    return probs, link.reshape(1), ent.reshape(1)


def kernel(x, edge_index, batch_idx, atom_emb, conv_Wl, conv_bl, conv_Wr,
           bn_gamma, bn_beta, pool_Wl, pool_bl, pool_Wr,
           assign_Wl, assign_bl, assign_Wr, lin_W, lin_b):
    src = edge_index[0].astype(jnp.int32)
    dst = edge_index[1].astype(jnp.int32)
    pad = _EPAD - _E
    srcp = jnp.concatenate([src, jnp.full((pad,), _N, jnp.int32)])
    dstp = jnp.concatenate([dst, jnp.zeros((pad,), jnp.int32)])
    adj2 = _adj_sc_kernel()(srcp, dstp).reshape(_NC, _SH)
    adjp = adj2[:, : _N * _PN].reshape(_NC, _N, _PN)
    xo = x.astype(jnp.int32) + jnp.asarray(_OFFSETS)[None, :]
    return _dense_tc(xo, adjp, atom_emb, conv_Wl, conv_bl,
                     conv_Wr, bn_gamma, bn_beta, assign_Wl, assign_bl,
                     assign_Wr, lin_W, lin_b)


# speedup vs baseline: 32.1618x; 1.8689x over previous
import functools

import jax
import jax.numpy as jnp
import numpy as np
from jax import lax
from jax.experimental import pallas as pl
from jax.experimental.pallas import tpu as pltpu
from jax.experimental.pallas import tpu_sc as plsc

_N = 10000
_E = 320000
_G = 400
_PN = 25
_C = 128
_CL = 7

_OFFSETS = np.concatenate([[0], np.cumsum([119, 5, 12, 12, 10, 6, 6, 2, 2])[:-1]]).astype(np.int32)
_TBL = 174

_NC, _NS = 2, 16
_EPT = 10240
_ROWS = _EPT // 128
_EPAD = _EPT * _NC * _NS
_SH = 250112
_SLC = _SH // _NS


def _adj_body(src_hbm, dst_hbm, out_hbm, src_v, dst_v, idx_v, ones_v, z_v, acc_sh):
    c = lax.axis_index("c")
    s = lax.axis_index("s")
    wid = c * _NS + s
    base = wid * _EPT

    zero16 = jnp.zeros((16,), jnp.float32)

    def zbody(j, carry):
        z_v[pl.ds(j * 16, 16)] = zero16
        return carry

    lax.fori_loop(0, _SLC // 16, zbody, 0)
    pltpu.sync_copy(z_v, acc_sh.at[pl.ds(s * _SLC, _SLC)])

    pltpu.sync_copy(src_hbm.at[pl.ds(base, _EPT)], src_v)
    pltpu.sync_copy(dst_hbm.at[pl.ds(base, _EPT)], dst_v)

    ones16 = jnp.ones((16,), jnp.float32)
    for j in range(8):
        ones_v[pl.ds(j * 16, 16)] = ones16

    def cbody(v, carry):
        sv = src_v[pl.ds(v * 16, 16)]
        dv = dst_v[pl.ds(v * 16, 16)]
        iv = sv * 25 + lax.rem(dv, 25)
        idx_v[v // 8, pl.ds(lax.rem(v, 8) * 16, 16)] = iv
        return carry

    lax.fori_loop(0, _ROWS * 8, cbody, 0)

    plsc.subcore_barrier()

    def sbody(r, carry):
        pltpu.sync_copy(ones_v, acc_sh.at[idx_v.at[r]], add=True)
        return carry

    lax.fori_loop(0, _ROWS, sbody, 0)

    plsc.subcore_barrier()
    pltpu.sync_copy(acc_sh.at[pl.ds(s * _SLC, _SLC)], z_v)
    pltpu.sync_copy(z_v, out_hbm.at[pl.ds(c * _SH + s * _SLC, _SLC)])


@functools.cache
def _adj_sc_kernel():
    return pl.kernel(
        _adj_body,
        out_type=jax.ShapeDtypeStruct((_NC * _SH,), jnp.float32),
        scratch_types=[
            pltpu.VMEM((_EPT,), jnp.int32),
            pltpu.VMEM((_EPT,), jnp.int32),
            pltpu.VMEM((_ROWS, 128), jnp.int32),
            pltpu.VMEM((128,), jnp.float32),
            pltpu.VMEM((_SLC,), jnp.float32),
            pltpu.VMEM_SHARED((_SH,), jnp.float32),
        ],
        mesh=plsc.VectorSubcoreMesh(core_axis_name="c", subcore_axis_name="s",
                                    num_cores=_NC, num_subcores=_NS),
    )


_GB = 16
_NB = _GB * _PN
_GQ = _GB * _CL
_NBLK = _G // _GB
_INV_SQRT = float(1.0 / np.sqrt(1.0 + 1e-05))


def _dot(a, b):
    return lax.dot_general(a, b, (((1,), (0,)), ((), ())),
                           precision=lax.Precision.DEFAULT,
                           preferred_element_type=jnp.float32)


def _dot_nt(a, b):
    return lax.dot_general(a, b, (((1,), (1,)), ((), ())),
                           precision=lax.Precision.DEFAULT,
                           preferred_element_type=jnp.float32)


def _dot_bf(a, b):
    return lax.dot_general(a.astype(jnp.bfloat16), b.astype(jnp.bfloat16),
                           (((1,), (0,)), ((), ())),
                           preferred_element_type=jnp.float32)


def _tc_body(x_ref, adj_ref, emb_ref, Wl_ref, bl_ref, Wr_ref, gam_ref, bet_ref,
             aWl_ref, abl_ref, aWr_ref, lW_ref, lb_ref,
             probs_ref, link_ref, ent_ref):
    i = pl.program_id(0)

    xi = x_ref[...]
    t_iota = lax.broadcasted_iota(jnp.int32, (_NB, _TBL), 1)
    oh = jnp.zeros((_NB, _TBL), jnp.float32)
    for f in range(9):
        oh = oh + (t_iota == xi[:, f][:, None]).astype(jnp.float32)
    h = _dot(oh, emb_ref[...])

    adjA = adj_ref[0] + adj_ref[1]
    r2 = lax.broadcasted_iota(jnp.int32, (_NB, _NB), 0)
    c2 = lax.broadcasted_iota(jnp.int32, (_NB, _NB), 1)
    samegraph = (r2 // _PN == c2 // _PN).astype(jnp.float32)
    rr = lax.broadcasted_iota(jnp.int32, (_NB, _PN), 0)
    kk = lax.broadcasted_iota(jnp.int32, (_NB, _PN), 1)
    oh_dmod = (lax.rem(rr, _PN) == kk).astype(jnp.float32)
    bdT = _dot_nt(oh_dmod, adjA) * samegraph
    denom = jnp.maximum(jnp.sum(bdT, axis=1, keepdims=True), 1.0)

    for li in range(2):
        res = h
        mean = _dot(bdT, h) / denom
        h = _dot(mean, Wl_ref[li]) + bl_ref[li][None, :] + _dot(h, Wr_ref[li])
        h = gam_ref[li][None, :] * h * _INV_SQRT + bet_ref[li][None, :]
        h = jnp.maximum(h, 0.0) + res

    meanA = _dot(bdT, h) / denom
    sr = _dot(meanA, aWl_ref[...]) + abl_ref[...] + _dot(h, aWr_ref[...])
    sr = sr - jnp.max(sr, axis=1, keepdims=True)
    se = jnp.exp(sr)
    S = se / jnp.sum(se, axis=1, keepdims=True)

    bdA = _dot_nt(adjA, oh_dmod)
    Q = _dot_nt(S, S)
    dlt = (bdA - Q) * samegraph
    linkpart = jnp.sum(dlt * dlt, keepdims=True).reshape(1, 1)
    entpart = jnp.sum(-S * jnp.log(S + 1e-15), keepdims=True).reshape(1, 1)

    qq = lax.broadcasted_iota(jnp.int32, (_GQ, _CL), 0)
    kk7 = lax.broadcasted_iota(jnp.int32, (_GQ, _CL), 1)
    oh7 = (lax.rem(qq, _CL) == kk7).astype(jnp.float32)
    qr = lax.broadcasted_iota(jnp.int32, (_GQ, _NB), 0)
    rc = lax.broadcasted_iota(jnp.int32, (_GQ, _NB), 1)
    qmask = (qr // _CL == rc // _PN).astype(jnp.float32)
    P = _dot_nt(oh7, S) * qmask
    h2 = _dot(P, h)

    q2r = lax.broadcasted_iota(jnp.int32, (_GQ, _GQ), 0)
    q2c = lax.broadcasted_iota(jnp.int32, (_GQ, _GQ), 1)
    Mm = (q2r // _CL == q2c // _CL).astype(jnp.float32) * (1.0 / _CL)
    for li in (2, 3):
        res = h2
        mean2 = _dot(Mm, h2)
        h2 = _dot_bf(mean2, Wl_ref[li]) + bl_ref[li][None, :] + _dot_bf(h2, Wr_ref[li])
        h2 = gam_ref[li][None, :] * h2 * _INV_SQRT + bet_ref[li][None, :]
        h2 = h2 + res

    gr = lax.broadcasted_iota(jnp.int32, (_GB, _GQ), 0)
    gc = lax.broadcasted_iota(jnp.int32, (_GB, _GQ), 1)
    R = (gc // _CL == gr).astype(jnp.float32) * (1.0 / _CL)
    hg = _dot(R, h2)
    logit = _dot(hg, lW_ref[...]) + lb_ref[...]
    probs_ref[...] = 1.0 / (1.0 + jnp.exp(-logit))

    @pl.when(i == 0)
    def _():
        link_ref[...] = jnp.zeros((1, 1), jnp.float32)
        ent_ref[...] = jnp.zeros((1, 1), jnp.float32)

    link_ref[...] += linkpart
    ent_ref[...] += entpart

    @pl.when(i == _NBLK - 1)
    def _():
        link_ref[...] = jnp.sqrt(link_ref[...]) * (1.0 / float(_G * _PN * _PN))
        ent_ref[...] = ent_ref[...] * (1.0 / float(_N))


def _dense_tc(x, adjp, atom_emb, conv_Wl, conv_bl, conv_Wr, bn_gamma, bn_beta,
              assign_Wl, assign_bl, assign_Wr, lin_W, lin_b):
    full = lambda shp: pl.BlockSpec(shp, lambda i: tuple(0 for _ in shp))
    probs, link, ent = pl.pallas_call(
        _tc_body,
        grid=(_NBLK,),
        in_specs=[
            pl.BlockSpec((_NB, 9), lambda i: (i, 0)),
            pl.BlockSpec((_NC, _NB, _PN), lambda i: (0, i, 0)),
            full((_TBL, _C)),
            full((4, _C, _C)),
            full((4, _C)),
            full((4, _C, _C)),
            full((4, _C)),
            full((4, _C)),
            full((_C, _CL)),
            full((1, _CL)),
            full((_C, _CL)),
            full((_C, 1)),
            full((1, 1)),
        ],
        out_specs=[
            pl.BlockSpec((_GB, 1), lambda i: (i, 0)),
            pl.BlockSpec((1, 1), lambda i: (0, 0)),
            pl.BlockSpec((1, 1), lambda i: (0, 0)),
        ],
        out_shape=[
            jax.ShapeDtypeStruct((_G, 1), jnp.float32),
            jax.ShapeDtypeStruct((1, 1), jnp.float32),
            jax.ShapeDtypeStruct((1, 1), jnp.float32),
        ],
    )(x, adjp, atom_emb, conv_Wl, conv_bl, conv_Wr, bn_gamma, bn_beta,
      assign_Wl, assign_bl.reshape(1, _CL), assign_Wr, lin_W, lin_b.reshape(1, 1))
    return probs, link.reshape(1), ent.reshape(1)


def kernel(x, edge_index, batch_idx, atom_emb, conv_Wl, conv_bl, conv_Wr,
           bn_gamma, bn_beta, pool_Wl, pool_bl, pool_Wr,
           assign_Wl, assign_bl, assign_Wr, lin_W, lin_b):
    src = edge_index[0].astype(jnp.int32)
    dst = edge_index[1].astype(jnp.int32)
    pad = _EPAD - _E
    srcp = jnp.concatenate([src, jnp.full((pad,), _N, jnp.int32)])
    dstp = jnp.concatenate([dst, jnp.zeros((pad,), jnp.int32)])
    adj2 = _adj_sc_kernel()(srcp, dstp).reshape(_NC, _SH)
    adjp = adj2[:, : _N * _PN].reshape(_NC, _N, _PN)
    xo = x.astype(jnp.int32) + jnp.asarray(_OFFSETS)[None, :]
    return _dense_tc(xo, adjp, atom_emb, conv_Wl, conv_bl,
                     conv_Wr, bn_gamma, bn_beta, assign_Wl, assign_bl,
                     assign_Wr, lin_W, lin_b)
